# baseline (speedup 1.0000x reference)
import functools

import jax
import jax.numpy as jnp
from jax.experimental import pallas as pl
from jax.experimental.pallas import tpu as pltpu

_BN_EPS = 1e-5
_VMEM_LIMIT = 48 * 1024 * 1024


def _conv_tail(a, w_ref, b_ref, y_ref, st_ref, *, th, wo, cin, cout):
    a3 = jnp.concatenate([a[:, kx:kx + wo, :] for kx in range(3)], axis=2)
    a9 = jnp.concatenate([a3[ky:ky + th] for ky in range(3)], axis=2)
    acc = jnp.dot(a9.reshape(th * wo, 9 * cin), w_ref[...],
                  preferred_element_type=jnp.float32)
    acc = acc + b_ref[...]
    y_ref[...] = acc.reshape(th, wo, cout).astype(y_ref.dtype)
    st_ref[0:1, :] = jnp.sum(acc, axis=0, keepdims=True)
    st_ref[1:2, :] = jnp.sum(acc * acc, axis=0, keepdims=True)


def _wilv(e, o, rows, wo, cin):
    return jnp.stack([e, o], axis=2).reshape(rows, wo, cin)


def _unpool_conv_kernel(xm_ref, im_ref, xt_ref, it_ref, xb_ref, ib_ref,
                        w_ref, b_ref, y_ref, st_ref, *, th, wi, cin, cout):
    hb = th // 2
    wo = 2 * wi
    i = pl.program_id(1)
    nhb = pl.num_programs(1)
    zero = jnp.zeros((), jnp.bfloat16)

    xm_c = xm_ref[...].astype(jnp.bfloat16)
    im_c = im_ref[...]
    xm = jnp.stack([jnp.transpose(xm_c[:, r, :]) for r in range(hb)], axis=0)
    im = jnp.stack([jnp.transpose(im_c[:, r, :]) for r in range(hb)], axis=0)
    rows = jax.lax.broadcasted_iota(jnp.int32, (hb, wi, cin), 0) + i * hb
    cols2 = 2 * jax.lax.broadcasted_iota(jnp.int32, (hb, wi, cin), 1)
    planes = []
    for dr in range(2):
        base = (2 * rows + dr) * wo + cols2
        e = jnp.where(im == base, xm, zero)
        o = jnp.where(im == base + 1, xm, zero)
        planes.append(_wilv(e, o, hb, wo, cin))
    um = jnp.stack(planes, axis=1).reshape(th, wo, cin)

    cols2h = 2 * jax.lax.broadcasted_iota(jnp.int32, (1, wi, cin), 1)
    xt = jnp.transpose(xt_ref[:, 7, :]).astype(jnp.bfloat16)[None]
    it = jnp.transpose(it_ref[:, 7, :])[None]
    bt = (2 * (i * hb - 1) + 1) * wo + cols2h
    ut = _wilv(jnp.where(it == bt, xt, zero),
               jnp.where(it == bt + 1, xt, zero), 1, wo, cin)
    ut = jnp.where(i == 0, zero, ut)
    xb = jnp.transpose(xb_ref[:, 0, :]).astype(jnp.bfloat16)[None]
    ib = jnp.transpose(ib_ref[:, 0, :])[None]
    bb = (2 * ((i + 1) * hb)) * wo + cols2h
    ub = _wilv(jnp.where(ib == bb, xb, zero),
               jnp.where(ib == bb + 1, xb, zero), 1, wo, cin)
    ub = jnp.where(i == nhb - 1, zero, ub)

    a = jnp.concatenate([ut, um, ub], axis=0)
    zc = jnp.zeros((th + 2, 1, cin), jnp.bfloat16)
    a = jnp.concatenate([zc, a, zc], axis=1)
    _conv_tail(a, w_ref, b_ref, y_ref, st_ref, th=th, wo=wo, cin=cin,
               cout=cout)


def _affine_conv_kernel(ym_ref, yt_ref, yb_ref, sc_ref, sh_ref,
                        w_ref, b_ref, y_ref, st_ref, *, th, wo, cin, cout):
    i = pl.program_id(1)
    nhb = pl.num_programs(1)
    sc = sc_ref[...]
    sh = sh_ref[...]

    def af(v):
        return jnp.maximum(v.astype(jnp.float32) * sc + sh, 0.0)

    mid = af(ym_ref[...])
    top = jnp.where(i == 0, 0.0, af(yt_ref[...]))
    bot = jnp.where(i == nhb - 1, 0.0, af(yb_ref[...]))
    a = jnp.concatenate([top, mid, bot], axis=0).astype(jnp.bfloat16)
    zc = jnp.zeros((th + 2, 1, cin), jnp.bfloat16)
    a = jnp.concatenate([zc, a, zc], axis=1)
    _conv_tail(a, w_ref, b_ref, y_ref, st_ref, th=th, wo=wo, cin=cin,
               cout=cout)


def _final_bn_relu_kernel(y_ref, sc_ref, sh_ref, o_ref, *, th, wo, cout):
    z = jnp.maximum(
        y_ref[...].astype(jnp.float32) * sc_ref[...] + sh_ref[...], 0.0)
    zt = jnp.transpose(z.reshape(th * wo, cout))
    o_ref[...] = zt.reshape(cout, th, wo)


def _combine_stats(st, nb, total):
    s1 = st[..., 0, :]
    s2 = st[..., 1, :]
    mean_b = s1 / nb
    m2_b = jnp.maximum(s2 - s1 * mean_b, 0.0)
    mean = jnp.mean(mean_b, axis=(0, 1))
    m2 = (jnp.sum(m2_b, axis=(0, 1))
          + nb * jnp.sum((mean_b - mean) ** 2, axis=(0, 1)))
    return mean, m2 / total


def _affine(g, beta, mean, var, c):
    inv = jax.lax.rsqrt(var + _BN_EPS)
    scale = (g * inv).reshape(1, c).astype(jnp.float32)
    shift = (beta - mean * g * inv).reshape(1, c).astype(jnp.float32)
    return scale, shift


def kernel(x, ind, w9_0, b2_0, g_0, beta_0, w9_1, b2_1, g_1, beta_1):
    N, cin, H, W = x.shape
    Ho, Wo = 2 * H, 2 * W
    c1 = int(w9_0.shape[2])
    c2 = int(w9_1.shape[2])
    th = min(32, Ho)
    hb = th // 2
    nhb = Ho // th

    w3_0 = w9_0.reshape(9 * cin, c1)
    w3_1 = w9_1.reshape(9 * c1, c2)

    cp = pltpu.CompilerParams(
        dimension_semantics=("parallel", "arbitrary"),
        vmem_limit_bytes=_VMEM_LIMIT)

    row_spec = lambda bs, c: pl.BlockSpec((None, bs, W, c),
                                          lambda n, i: (n, i, 0, 0))
    k1 = functools.partial(_unpool_conv_kernel, th=th, wi=W, cin=cin, cout=c1)
    flops1 = 2 * N * Ho * Wo * 9 * cin * c1
    y1, st1 = pl.pallas_call(
        k1,
        grid=(N, nhb),
        in_specs=[
            pl.BlockSpec((None, cin, hb, W), lambda n, i: (n, 0, i, 0)),
            pl.BlockSpec((None, cin, hb, W), lambda n, i: (n, 0, i, 0)),
            pl.BlockSpec((None, cin, 8, W),
                         lambda n, i: (n, 0, jnp.maximum((i * hb - 1) // 8, 0), 0)),
            pl.BlockSpec((None, cin, 8, W),
                         lambda n, i: (n, 0, jnp.maximum((i * hb - 1) // 8, 0), 0)),
            pl.BlockSpec((None, cin, 8, W),
                         lambda n, i: (n, 0, jnp.minimum(((i + 1) * hb) // 8, H // 8 - 1), 0)),
            pl.BlockSpec((None, cin, 8, W),
                         lambda n, i: (n, 0, jnp.minimum(((i + 1) * hb) // 8, H // 8 - 1), 0)),
            pl.BlockSpec((9 * cin, c1), lambda n, i: (0, 0)),
            pl.BlockSpec((1, c1), lambda n, i: (0, 0)),
        ],
        out_shape=(
            jax.ShapeDtypeStruct((N, Ho, Wo, c1), jnp.bfloat16),
            jax.ShapeDtypeStruct((N, nhb, 2, c1), jnp.float32),
        ),
        out_specs=(
            pl.BlockSpec((None, th, Wo, c1), lambda n, i: (n, i, 0, 0)),
            pl.BlockSpec((None, None, 2, c1), lambda n, i: (n, i, 0, 0)),
        ),
        compiler_params=cp,
        cost_estimate=pl.CostEstimate(
            flops=flops1, transcendentals=0,
            bytes_accessed=(N * H * W * cin * 8 + N * Ho * Wo * c1 * 2)),
    )(x, ind, x, ind, x, ind, w3_0, b2_0)

    mean1, var1 = _combine_stats(st1, th * Wo, N * Ho * Wo)
    scale1, shift1 = _affine(g_0, beta_0, mean1, var1, c1)

    k2 = functools.partial(_affine_conv_kernel, th=th, wo=Wo, cin=c1, cout=c2)
    flops2 = 2 * N * Ho * Wo * 9 * c1 * c2
    vspec = lambda c: pl.BlockSpec((1, c), lambda n, i: (0, 0))
    y2, st2 = pl.pallas_call(
        k2,
        grid=(N, nhb),
        in_specs=[
            pl.BlockSpec((None, th, Wo, c1), lambda n, i: (n, i, 0, 0)),
            pl.BlockSpec((None, 1, Wo, c1),
                         lambda n, i: (n, jnp.maximum(i * th - 1, 0), 0, 0)),
            pl.BlockSpec((None, 1, Wo, c1),
                         lambda n, i: (n, jnp.minimum((i + 1) * th, Ho - 1), 0, 0)),
            vspec(c1),
            vspec(c1),
            pl.BlockSpec((9 * c1, c2), lambda n, i: (0, 0)),
            vspec(c2),
        ],
        out_shape=(
            jax.ShapeDtypeStruct((N, Ho, Wo, c2), jnp.bfloat16),
            jax.ShapeDtypeStruct((N, nhb, 2, c2), jnp.float32),
        ),
        out_specs=(
            pl.BlockSpec((None, th, Wo, c2), lambda n, i: (n, i, 0, 0)),
            pl.BlockSpec((None, None, 2, c2), lambda n, i: (n, i, 0, 0)),
        ),
        compiler_params=cp,
        cost_estimate=pl.CostEstimate(
            flops=flops2, transcendentals=0,
            bytes_accessed=(N * Ho * Wo * c1 * 2 + N * Ho * Wo * c2 * 2)),
    )(y1, y1, y1, scale1, shift1, w3_1, b2_1)

    mean2, var2 = _combine_stats(st2, th * Wo, N * Ho * Wo)
    scale2, shift2 = _affine(g_1, beta_1, mean2, var2, c2)

    th3 = min(32, Ho)
    nh3 = Ho // th3
    k3 = functools.partial(_final_bn_relu_kernel, th=th3, wo=Wo, cout=c2)
    out = pl.pallas_call(
        k3,
        grid=(N, nh3),
        in_specs=[
            pl.BlockSpec((None, th3, Wo, c2), lambda n, i: (n, i, 0, 0)),
            vspec(c2),
            vspec(c2),
        ],
        out_shape=jax.ShapeDtypeStruct((N, c2, Ho, Wo), jnp.float32),
        out_specs=pl.BlockSpec((None, c2, th3, Wo), lambda n, i: (n, 0, i, 0)),
        compiler_params=cp,
        cost_estimate=pl.CostEstimate(
            flops=2 * N * Ho * Wo * c2, transcendentals=0,
            bytes_accessed=N * Ho * Wo * c2 * 6),
    )(y2, scale2, shift2)

    return out


# speedup vs baseline: 1.3918x; 1.3918x over previous
import functools

import jax
import jax.numpy as jnp
from jax.experimental import pallas as pl
from jax.experimental.pallas import tpu as pltpu

_BN_EPS = 1e-5
_VMEM_LIMIT = 48 * 1024 * 1024


def _conv_tail(a, w_ref, b_ref, y_ref, st_ref, *, th, wo, cin, cout):
    a3 = jnp.concatenate([a[:, kx:kx + wo, :] for kx in range(3)], axis=2)
    a9 = jnp.concatenate([a3[ky:ky + th] for ky in range(3)], axis=2)
    acc = jnp.dot(a9.reshape(th * wo, 9 * cin), w_ref[...],
                  preferred_element_type=jnp.float32)
    acc = acc + b_ref[...]
    y_ref[...] = acc.reshape(th, wo, cout).astype(y_ref.dtype)
    st_ref[0:1, :] = jnp.sum(acc, axis=0, keepdims=True)
    st_ref[1:2, :] = jnp.sum(acc * acc, axis=0, keepdims=True)


def _wilv(e, o, rows, wo, cin):
    return jnp.stack([e, o], axis=2).reshape(rows, wo, cin)


def _unpool_conv_kernel(xm_ref, im_ref, xt_ref, it_ref, xb_ref, ib_ref,
                        w_ref, b_ref, y_ref, st_ref, *, th, wi, cin, cout):
    hb = th // 2
    wo = 2 * wi
    i = pl.program_id(1)
    nhb = pl.num_programs(1)
    zero = jnp.zeros((), jnp.bfloat16)

    xm = xm_ref[...].astype(jnp.bfloat16)
    im = im_ref[...]
    rows = jax.lax.broadcasted_iota(jnp.int32, (hb, wi, cin), 0) + i * hb
    cols2 = 2 * jax.lax.broadcasted_iota(jnp.int32, (hb, wi, cin), 1)
    planes = []
    for dr in range(2):
        base = (2 * rows + dr) * wo + cols2
        e = jnp.where(im == base, xm, zero)
        o = jnp.where(im == base + 1, xm, zero)
        planes.append(_wilv(e, o, hb, wo, cin))
    um = jnp.stack(planes, axis=1).reshape(th, wo, cin)

    cols2h = 2 * jax.lax.broadcasted_iota(jnp.int32, (1, wi, cin), 1)
    xt = xt_ref[...].astype(jnp.bfloat16)
    bt = (2 * (i * hb - 1) + 1) * wo + cols2h
    ut = _wilv(jnp.where(it_ref[...] == bt, xt, zero),
               jnp.where(it_ref[...] == bt + 1, xt, zero), 1, wo, cin)
    ut = jnp.where(i == 0, zero, ut)
    xb = xb_ref[...].astype(jnp.bfloat16)
    bb = (2 * ((i + 1) * hb)) * wo + cols2h
    ub = _wilv(jnp.where(ib_ref[...] == bb, xb, zero),
               jnp.where(ib_ref[...] == bb + 1, xb, zero), 1, wo, cin)
    ub = jnp.where(i == nhb - 1, zero, ub)

    a = jnp.concatenate([ut, um, ub], axis=0)
    zc = jnp.zeros((th + 2, 1, cin), jnp.bfloat16)
    a = jnp.concatenate([zc, a, zc], axis=1)
    _conv_tail(a, w_ref, b_ref, y_ref, st_ref, th=th, wo=wo, cin=cin,
               cout=cout)


def _affine_conv_kernel(ym_ref, yt_ref, yb_ref, sc_ref, sh_ref,
                        w_ref, b_ref, y_ref, st_ref, *, th, wo, cin, cout):
    i = pl.program_id(1)
    nhb = pl.num_programs(1)
    sc = sc_ref[...]
    sh = sh_ref[...]

    def af(v):
        return jnp.maximum(v.astype(jnp.float32) * sc + sh, 0.0)

    mid = af(ym_ref[...])
    top = jnp.where(i == 0, 0.0, af(yt_ref[...]))
    bot = jnp.where(i == nhb - 1, 0.0, af(yb_ref[...]))
    a = jnp.concatenate([top, mid, bot], axis=0).astype(jnp.bfloat16)
    zc = jnp.zeros((th + 2, 1, cin), jnp.bfloat16)
    a = jnp.concatenate([zc, a, zc], axis=1)
    _conv_tail(a, w_ref, b_ref, y_ref, st_ref, th=th, wo=wo, cin=cin,
               cout=cout)


def _final_bn_relu_kernel(y_ref, sc_ref, sh_ref, o_ref, *, th, wo, cout):
    z = jnp.maximum(
        y_ref[...].astype(jnp.float32) * sc_ref[...] + sh_ref[...], 0.0)
    zt = jnp.transpose(z.reshape(th * wo, cout))
    o_ref[...] = zt.reshape(cout, th, wo)


def _combine_stats(st, nb, total):
    s1 = st[..., 0, :]
    s2 = st[..., 1, :]
    mean_b = s1 / nb
    m2_b = jnp.maximum(s2 - s1 * mean_b, 0.0)
    mean = jnp.mean(mean_b, axis=(0, 1))
    m2 = (jnp.sum(m2_b, axis=(0, 1))
          + nb * jnp.sum((mean_b - mean) ** 2, axis=(0, 1)))
    return mean, m2 / total


def _affine(g, beta, mean, var, c):
    inv = jax.lax.rsqrt(var + _BN_EPS)
    scale = (g * inv).reshape(1, c).astype(jnp.float32)
    shift = (beta - mean * g * inv).reshape(1, c).astype(jnp.float32)
    return scale, shift


def kernel(x, ind, w9_0, b2_0, g_0, beta_0, w9_1, b2_1, g_1, beta_1):
    N, cin, H, W = x.shape
    Ho, Wo = 2 * H, 2 * W
    c1 = int(w9_0.shape[2])
    c2 = int(w9_1.shape[2])
    th = min(32, Ho)
    hb = th // 2
    nhb = Ho // th

    w3_0 = w9_0.reshape(9 * cin, c1)
    w3_1 = w9_1.reshape(9 * c1, c2)
    xv = jnp.transpose(x, (0, 2, 3, 1))
    iv = jnp.transpose(ind, (0, 2, 3, 1))

    cp = pltpu.CompilerParams(
        dimension_semantics=("parallel", "arbitrary"),
        vmem_limit_bytes=_VMEM_LIMIT)

    row_spec = lambda bs, c: pl.BlockSpec((None, bs, W, c),
                                          lambda n, i: (n, i, 0, 0))
    k1 = functools.partial(_unpool_conv_kernel, th=th, wi=W, cin=cin, cout=c1)
    flops1 = 2 * N * Ho * Wo * 9 * cin * c1
    y1, st1 = pl.pallas_call(
        k1,
        grid=(N, nhb),
        in_specs=[
            pl.BlockSpec((None, hb, W, cin), lambda n, i: (n, i, 0, 0)),
            pl.BlockSpec((None, hb, W, cin), lambda n, i: (n, i, 0, 0)),
            pl.BlockSpec((None, 1, W, cin),
                         lambda n, i: (n, jnp.maximum(i * hb - 1, 0), 0, 0)),
            pl.BlockSpec((None, 1, W, cin),
                         lambda n, i: (n, jnp.maximum(i * hb - 1, 0), 0, 0)),
            pl.BlockSpec((None, 1, W, cin),
                         lambda n, i: (n, jnp.minimum((i + 1) * hb, H - 1), 0, 0)),
            pl.BlockSpec((None, 1, W, cin),
                         lambda n, i: (n, jnp.minimum((i + 1) * hb, H - 1), 0, 0)),
            pl.BlockSpec((9 * cin, c1), lambda n, i: (0, 0)),
            pl.BlockSpec((1, c1), lambda n, i: (0, 0)),
        ],
        out_shape=(
            jax.ShapeDtypeStruct((N, Ho, Wo, c1), jnp.bfloat16),
            jax.ShapeDtypeStruct((N, nhb, 2, c1), jnp.float32),
        ),
        out_specs=(
            pl.BlockSpec((None, th, Wo, c1), lambda n, i: (n, i, 0, 0)),
            pl.BlockSpec((None, None, 2, c1), lambda n, i: (n, i, 0, 0)),
        ),
        compiler_params=cp,
        cost_estimate=pl.CostEstimate(
            flops=flops1, transcendentals=0,
            bytes_accessed=(N * H * W * cin * 8 + N * Ho * Wo * c1 * 2)),
    )(xv, iv, xv, iv, xv, iv, w3_0, b2_0)

    mean1, var1 = _combine_stats(st1, th * Wo, N * Ho * Wo)
    scale1, shift1 = _affine(g_0, beta_0, mean1, var1, c1)

    k2 = functools.partial(_affine_conv_kernel, th=th, wo=Wo, cin=c1, cout=c2)
    flops2 = 2 * N * Ho * Wo * 9 * c1 * c2
    vspec = lambda c: pl.BlockSpec((1, c), lambda n, i: (0, 0))
    y2, st2 = pl.pallas_call(
        k2,
        grid=(N, nhb),
        in_specs=[
            pl.BlockSpec((None, th, Wo, c1), lambda n, i: (n, i, 0, 0)),
            pl.BlockSpec((None, 1, Wo, c1),
                         lambda n, i: (n, jnp.maximum(i * th - 1, 0), 0, 0)),
            pl.BlockSpec((None, 1, Wo, c1),
                         lambda n, i: (n, jnp.minimum((i + 1) * th, Ho - 1), 0, 0)),
            vspec(c1),
            vspec(c1),
            pl.BlockSpec((9 * c1, c2), lambda n, i: (0, 0)),
            vspec(c2),
        ],
        out_shape=(
            jax.ShapeDtypeStruct((N, Ho, Wo, c2), jnp.bfloat16),
            jax.ShapeDtypeStruct((N, nhb, 2, c2), jnp.float32),
        ),
        out_specs=(
            pl.BlockSpec((None, th, Wo, c2), lambda n, i: (n, i, 0, 0)),
            pl.BlockSpec((None, None, 2, c2), lambda n, i: (n, i, 0, 0)),
        ),
        compiler_params=cp,
        cost_estimate=pl.CostEstimate(
            flops=flops2, transcendentals=0,
            bytes_accessed=(N * Ho * Wo * c1 * 2 + N * Ho * Wo * c2 * 2)),
    )(y1, y1, y1, scale1, shift1, w3_1, b2_1)

    mean2, var2 = _combine_stats(st2, th * Wo, N * Ho * Wo)
    scale2, shift2 = _affine(g_1, beta_1, mean2, var2, c2)

    th3 = min(32, Ho)
    nh3 = Ho // th3
    k3 = functools.partial(_final_bn_relu_kernel, th=th3, wo=Wo, cout=c2)
    out = pl.pallas_call(
        k3,
        grid=(N, nh3),
        in_specs=[
            pl.BlockSpec((None, th3, Wo, c2), lambda n, i: (n, i, 0, 0)),
            vspec(c2),
            vspec(c2),
        ],
        out_shape=jax.ShapeDtypeStruct((N, c2, Ho, Wo), jnp.float32),
        out_specs=pl.BlockSpec((None, c2, th3, Wo), lambda n, i: (n, 0, i, 0)),
        compiler_params=cp,
        cost_estimate=pl.CostEstimate(
            flops=2 * N * Ho * Wo * c2, transcendentals=0,
            bytes_accessed=N * Ho * Wo * c2 * 6),
    )(y2, scale2, shift2)

    return out
